# speedup vs baseline: 1.0776x; 1.0776x over previous
import functools

import jax
import jax.numpy as jnp
from jax import lax
from jax.experimental import pallas as pl
from jax.experimental.pallas import tpu as pltpu
from jax.experimental.pallas import tpu_sc as plsc

_NW = 32
_K = 128


@functools.lru_cache(maxsize=None)
def _make_gather(M, D, dtype):
    B_pad = _NW * M * _K
    mesh = plsc.VectorSubcoreMesh(core_axis_name="c", subcore_axis_name="s")

    @functools.partial(
        pl.kernel,
        mesh=mesh,
        out_type=jax.ShapeDtypeStruct((B_pad, D), dtype),
        scratch_types=[
            pltpu.VMEM((M, _K), jnp.int32),
            pltpu.VMEM((_K, D), dtype),
            pltpu.SemaphoreType.DMA,
        ],
    )
    def gather_kernel(table_hbm, idx_hbm, out_hbm, idx_v, rows_v, sem):
        wid = lax.axis_index("s") * 2 + lax.axis_index("c")
        base = wid * (M * _K)
        pltpu.sync_copy(idx_hbm.at[wid], idx_v)

        def body(j, carry):
            pltpu.async_copy(table_hbm.at[idx_v.at[j]], rows_v, sem).wait()
            pltpu.sync_copy(rows_v, out_hbm.at[pl.ds(base + j * _K, _K)])
            return carry

        lax.fori_loop(0, M, body, 0)

    return gather_kernel


def kernel(x_coarse, P):
    B = P.shape[0]
    D = x_coarse.shape[1]
    per = _NW * _K
    M = -(-B // per)
    B_pad = M * per
    idx = P.astype(jnp.int32)
    if B_pad != B:
        idx = jnp.pad(idx, (0, B_pad - B))
    idx = idx.reshape(_NW, M, _K)
    out = _make_gather(M, D, x_coarse.dtype)(x_coarse, idx)
    return out[:B] if B_pad != B else out


# speedup vs baseline: 1.2113x; 1.1241x over previous
import functools

import jax
import jax.numpy as jnp
from jax import lax
from jax.experimental import pallas as pl
from jax.experimental.pallas import tpu as pltpu
from jax.experimental.pallas import tpu_sc as plsc

_NW = 32
_K = 128
_NBUF = 5


@functools.lru_cache(maxsize=None)
def _make_gather(M, D, dtype):
    B_pad = _NW * M * _K
    P_OUTER = M // _NBUF
    mesh = plsc.VectorSubcoreMesh(core_axis_name="c", subcore_axis_name="s")

    @functools.partial(
        pl.kernel,
        mesh=mesh,
        out_type=jax.ShapeDtypeStruct((B_pad, D), dtype),
        scratch_types=[
            pltpu.VMEM((M, _K), jnp.int32),
            *[pltpu.VMEM((_K, D), dtype) for _ in range(_NBUF)],
            *[pltpu.SemaphoreType.DMA for _ in range(2 * _NBUF)],
        ],
    )
    def gather_kernel(table_hbm, idx_hbm, out_hbm, idx_v, *bufs_and_sems):
        rows = bufs_and_sems[:_NBUF]
        sg = bufs_and_sems[_NBUF:2 * _NBUF]
        sw = bufs_and_sems[2 * _NBUF:]
        wid = lax.axis_index("s") * 2 + lax.axis_index("c")
        base = wid * (M * _K)
        pltpu.sync_copy(idx_hbm.at[wid], idx_v)

        for b in range(_NBUF):
            pltpu.async_copy(table_hbm.at[idx_v.at[b]], rows[b], sg[b])

        def body(p, carry):
            for b in range(_NBUF):
                j = p * _NBUF + b
                pltpu.make_async_copy(table_hbm.at[idx_v.at[b]], rows[b],
                                      sg[b]).wait()
                w = pltpu.async_copy(
                    rows[b], out_hbm.at[pl.ds(base + j * _K, _K)], sw[b])
                w.wait()
                pltpu.async_copy(table_hbm.at[idx_v.at[j + _NBUF]], rows[b],
                                 sg[b])
            return carry

        lax.fori_loop(0, P_OUTER - 1, body, 0, unroll=False)

        writes = []
        for b in range(_NBUF):
            j = (P_OUTER - 1) * _NBUF + b
            pltpu.make_async_copy(table_hbm.at[idx_v.at[b]], rows[b],
                                  sg[b]).wait()
            writes.append(pltpu.async_copy(
                rows[b], out_hbm.at[pl.ds(base + j * _K, _K)], sw[b]))
        for w in writes:
            w.wait()

    return gather_kernel


def kernel(x_coarse, P):
    B = P.shape[0]
    D = x_coarse.shape[1]
    per = _NW * _K * _NBUF
    n_super = -(-B // per)
    M = n_super * _NBUF
    B_pad = _NW * M * _K
    idx = P.astype(jnp.int32)
    if B_pad != B:
        idx = jnp.pad(idx, (0, B_pad - B))
    idx = idx.reshape(_NW, M, _K)
    out = _make_gather(M, D, x_coarse.dtype)(x_coarse, idx)
    return out[:B] if B_pad != B else out


# speedup vs baseline: 1.2143x; 1.0025x over previous
import functools

import jax
import jax.numpy as jnp
from jax import lax
from jax.experimental import pallas as pl
from jax.experimental.pallas import tpu as pltpu
from jax.experimental.pallas import tpu_sc as plsc

_NS = 16
_K = 128
_NBUF = 5
_M0 = 40
_M1 = 10


@functools.lru_cache(maxsize=None)
def _make_gather(D, dtype):
    M_MAX = max(_M0, _M1)
    C = _NS * (_M0 + _M1)
    B_pad = C * _K
    mesh = plsc.VectorSubcoreMesh(core_axis_name="c", subcore_axis_name="s")

    @functools.partial(
        pl.kernel,
        mesh=mesh,
        out_type=jax.ShapeDtypeStruct((B_pad, D), dtype),
        scratch_types=[
            pltpu.VMEM((M_MAX * _K,), jnp.int32),
            *[pltpu.VMEM((_K, D), dtype) for _ in range(_NBUF)],
            *[pltpu.SemaphoreType.DMA for _ in range(2 * _NBUF)],
        ],
    )
    def gather_kernel(table_hbm, idx_hbm, out_hbm, idx_v, *bufs_and_sems):
        rows = bufs_and_sems[:_NBUF]
        sg = bufs_and_sems[_NBUF:2 * _NBUF]
        sw = bufs_and_sems[2 * _NBUF:]
        c = lax.axis_index("c")
        s = lax.axis_index("s")
        m = jnp.where(c == 0, _M0, _M1)
        start = jnp.where(c == 0, s * _M0, _NS * _M0 + s * _M1)
        base = start * _K
        pltpu.sync_copy(idx_hbm.at[pl.ds(base, M_MAX * _K)], idx_v)

        def g(j, b):
            return pltpu.async_copy(
                table_hbm.at[idx_v.at[pl.ds(j * _K, _K)]], rows[b], sg[b])

        def g_drain(j, b):
            pltpu.make_async_copy(
                table_hbm.at[idx_v.at[pl.ds(j * _K, _K)]], rows[b],
                sg[b]).wait()

        def w(j, b):
            return pltpu.async_copy(
                rows[b], out_hbm.at[pl.ds(base + j * _K, _K)], sw[b])

        for b in range(_NBUF):
            g(b, b)

        def body(p, carry):
            for b in range(_NBUF):
                j = p * _NBUF + b
                g_drain(j, b)
                w(j, b).wait()
                g(j + _NBUF, b)
            return carry

        lax.fori_loop(0, m // _NBUF - 1, body, 0)

        writes = []
        for b in range(_NBUF):
            j = m - _NBUF + b
            g_drain(j, b)
            writes.append(w(j, b))
        for wr in writes:
            wr.wait()

    return gather_kernel


def kernel(x_coarse, P):
    B = P.shape[0]
    D = x_coarse.shape[1]
    M_MAX = max(_M0, _M1)
    C = _NS * (_M0 + _M1)
    B_pad = C * _K
    idx = P.astype(jnp.int32)
    idx = jnp.pad(idx, (0, B_pad + M_MAX * _K - B))
    out = _make_gather(D, x_coarse.dtype)(x_coarse, idx)
    return out[:B] if B_pad != B else out


# speedup vs baseline: 3.2976x; 2.7156x over previous
import functools

import jax
import jax.numpy as jnp
from jax import lax
from jax.experimental import pallas as pl
from jax.experimental.pallas import tpu as pltpu
from jax.experimental.pallas import tpu_sc as plsc

_NS = 16
_NW = 32
_K = 128
_NBUF = 2


@functools.lru_cache(maxsize=None)
def _make_gather(M, V, D, dtype):
    B_pad = _NW * M * _K
    v_chunk = (V // _NS) // 8 * 8
    v_last_off = v_chunk * (_NS - 1)
    v_last = V - v_last_off
    mesh = plsc.VectorSubcoreMesh(core_axis_name="c", subcore_axis_name="s")

    @functools.partial(
        pl.kernel,
        mesh=mesh,
        out_type=jax.ShapeDtypeStruct((B_pad, D), dtype),
        scratch_types=[
            pltpu.VMEM_SHARED((V, D), dtype),
            pltpu.VMEM((M * _K,), jnp.int32),
            *[pltpu.VMEM((_K, D), dtype) for _ in range(_NBUF)],
            *[pltpu.SemaphoreType.DMA for _ in range(2 * _NBUF)],
        ],
    )
    def gather_kernel(table_hbm, idx_hbm, out_hbm, shared, idx_v,
                      *bufs_and_sems):
        rows = bufs_and_sems[:_NBUF]
        sg = bufs_and_sems[_NBUF:2 * _NBUF]
        sw = bufs_and_sems[2 * _NBUF:]
        c = lax.axis_index("c")
        s = lax.axis_index("s")
        wid = s * 2 + c
        base = wid * (M * _K)

        pltpu.sync_copy(idx_hbm.at[pl.ds(base, M * _K)], idx_v)

        @pl.when(s < _NS - 1)
        def _stage_main():
            pltpu.sync_copy(table_hbm.at[pl.ds(s * v_chunk, v_chunk)],
                            shared.at[pl.ds(s * v_chunk, v_chunk)])

        @pl.when(s == _NS - 1)
        def _stage_last():
            pltpu.sync_copy(table_hbm.at[pl.ds(v_last_off, v_last)],
                            shared.at[pl.ds(v_last_off, v_last)])

        plsc.subcore_barrier()

        def g(j, b):
            return pltpu.async_copy(
                shared.at[idx_v.at[pl.ds(j * _K, _K)]], rows[b], sg[b])

        def g_drain(j, b):
            pltpu.make_async_copy(
                shared.at[idx_v.at[pl.ds(j * _K, _K)]], rows[b],
                sg[b]).wait()

        def w(j, b):
            return pltpu.async_copy(
                rows[b], out_hbm.at[pl.ds(base + j * _K, _K)], sw[b])

        for b in range(_NBUF):
            g(b, b)

        def body(p, carry):
            for b in range(_NBUF):
                j = p * _NBUF + b
                g_drain(j, b)
                w(j, b).wait()
                g(j + _NBUF, b)
            return carry

        lax.fori_loop(0, M // _NBUF - 1, body, 0)

        writes = []
        for b in range(_NBUF):
            j = M - _NBUF + b
            g_drain(j, b)
            writes.append(w(j, b))
        for wr in writes:
            wr.wait()

    return gather_kernel


def kernel(x_coarse, P):
    B = P.shape[0]
    V = x_coarse.shape[0]
    D = x_coarse.shape[1]
    per = _NW * _K * _NBUF
    n_super = -(-B // per)
    M = n_super * _NBUF
    B_pad = _NW * M * _K
    idx = P.astype(jnp.int32)
    if B_pad != B:
        idx = jnp.pad(idx, (0, B_pad - B))
    out = _make_gather(M, V, D, x_coarse.dtype)(x_coarse, idx)
    return out[:B] if B_pad != B else out


# speedup vs baseline: 5.4988x; 1.6675x over previous
import functools

import jax
import jax.numpy as jnp
from jax import lax
from jax.experimental import pallas as pl
from jax.experimental.pallas import tpu as pltpu
from jax.experimental.pallas import tpu_sc as plsc

_NS = 16
_NW = 32
_K = 128
_NBUF = 2


@functools.lru_cache(maxsize=None)
def _make_gather(B, V, D, dtype):
    n_full = B // _K
    tail = B - n_full * _K
    m_lo = (n_full // _NW) // _NBUF * _NBUF
    rem = n_full - _NW * m_lo
    n_hi = rem // _NBUF
    m_hi = m_lo + _NBUF
    n_extra = rem - n_hi * _NBUF
    extra_start = n_hi * m_hi + (_NW - n_hi) * m_lo
    slab_hi = m_hi * _K
    slab_lo = m_lo * _K
    slab_last = slab_lo + n_extra * _K + tail
    slab_max = max(slab_hi, slab_last)

    v_chunk = (V // _NS) // 8 * 8
    v_last_off = v_chunk * (_NS - 1)
    v_last = V - v_last_off

    mesh = plsc.VectorSubcoreMesh(core_axis_name="c", subcore_axis_name="s")

    @functools.partial(
        pl.kernel,
        mesh=mesh,
        out_type=jax.ShapeDtypeStruct((B, D), dtype),
        scratch_types=[
            pltpu.VMEM_SHARED((V, D), dtype),
            pltpu.VMEM((slab_max,), jnp.int32),
            *[pltpu.VMEM((_K, D), dtype) for _ in range(_NBUF)],
            *[pltpu.SemaphoreType.DMA for _ in range(2 * _NBUF)],
        ],
    )
    def gather_kernel(table_hbm, idx_hbm, out_hbm, shared, idx_v,
                      *bufs_and_sems):
        rows = bufs_and_sems[:_NBUF]
        sg = bufs_and_sems[_NBUF:2 * _NBUF]
        sw = bufs_and_sems[2 * _NBUF:]
        c = lax.axis_index("c")
        s = lax.axis_index("s")
        wid = s * 2 + c
        m = jnp.where(wid < n_hi, m_hi, m_lo)
        start = jnp.where(wid < n_hi, wid * m_hi,
                          n_hi * m_hi + (wid - n_hi) * m_lo)
        base = start * _K

        @pl.when(wid < n_hi)
        def _stage_idx_hi():
            pltpu.sync_copy(idx_hbm.at[pl.ds(base, slab_hi)],
                            idx_v.at[pl.ds(0, slab_hi)])

        @pl.when(jnp.logical_and(wid >= n_hi, wid < _NW - 1))
        def _stage_idx_lo():
            pltpu.sync_copy(idx_hbm.at[pl.ds(base, slab_lo)],
                            idx_v.at[pl.ds(0, slab_lo)])

        @pl.when(wid == _NW - 1)
        def _stage_idx_last():
            pltpu.sync_copy(idx_hbm.at[pl.ds(base, slab_last)],
                            idx_v.at[pl.ds(0, slab_last)])

        @pl.when(s < _NS - 1)
        def _stage_main():
            pltpu.sync_copy(table_hbm.at[pl.ds(s * v_chunk, v_chunk)],
                            shared.at[pl.ds(s * v_chunk, v_chunk)])

        @pl.when(s == _NS - 1)
        def _stage_last():
            pltpu.sync_copy(table_hbm.at[pl.ds(v_last_off, v_last)],
                            shared.at[pl.ds(v_last_off, v_last)])

        plsc.subcore_barrier()

        def g(j, b):
            return pltpu.async_copy(
                shared.at[idx_v.at[pl.ds(j * _K, _K)]], rows[b], sg[b])

        def g_drain(j, b):
            pltpu.make_async_copy(
                shared.at[idx_v.at[pl.ds(j * _K, _K)]], rows[b],
                sg[b]).wait()

        def w(j, b):
            return pltpu.async_copy(
                rows[b], out_hbm.at[pl.ds(base + j * _K, _K)], sw[b])

        for b in range(_NBUF):
            g(b, b)

        def body(p, carry):
            for b in range(_NBUF):
                j = p * _NBUF + b
                g_drain(j, b)
                w(j, b).wait()
                g(j + _NBUF, b)
            return carry

        lax.fori_loop(0, m // _NBUF - 1, body, 0)

        writes = []
        for b in range(_NBUF):
            j = m - _NBUF + b
            g_drain(j, b)
            writes.append(w(j, b))
        for wr in writes:
            wr.wait()

        if n_extra or tail:
            @pl.when(wid == _NW - 1)
            def _finish():
                for t in range(n_extra):
                    off = slab_lo + t * _K
                    row0 = extra_start * _K + t * _K
                    pltpu.async_copy(
                        shared.at[idx_v.at[pl.ds(off, _K)]], rows[0],
                        sg[0]).wait()
                    pltpu.async_copy(
                        rows[0], out_hbm.at[pl.ds(row0, _K)], sw[0]).wait()
                if tail:
                    off = slab_lo + n_extra * _K
                    row0 = (extra_start + n_extra) * _K
                    pltpu.async_copy(
                        shared.at[idx_v.at[pl.ds(off, tail)]],
                        rows[0].at[pl.ds(0, tail)], sg[0]).wait()
                    pltpu.async_copy(
                        rows[0].at[pl.ds(0, tail)],
                        out_hbm.at[pl.ds(row0, tail)], sw[0]).wait()

    return gather_kernel


def kernel(x_coarse, P):
    B = P.shape[0]
    V, D = x_coarse.shape
    idx = P.astype(jnp.int32)
    return _make_gather(B, V, D, x_coarse.dtype)(x_coarse, idx)
